# baseline (speedup 1.0000x reference)
import jax
import jax.numpy as jnp
from jax.experimental import pallas as pl
from jax.experimental.pallas import tpu as pltpu

_NBUF = 4


def _gcn_kernel(x_ref, wgc_ref, wfc_ref, bgc_ref, bfc_ref, fadj_ref,
                out_ref, xb_ref, buf_ref, sem_ref):
    i = pl.program_id(0)
    nsteps = pl.num_programs(0)
    bm = buf_ref.shape[1]

    def copy_in(step, slot):
        pltpu.make_async_copy(
            fadj_ref.at[pl.ds(step * bm, bm), :],
            buf_ref.at[slot],
            sem_ref.at[slot],
        ).start()

    @pl.when(i == 0)
    def _():
        xb_ref[...] = x_ref[...].astype(jnp.bfloat16)
        for s in range(_NBUF - 1):
            copy_in(s, s)

    nxt = i + _NBUF - 1

    @pl.when(nxt < nsteps)
    def _():
        copy_in(nxt, jax.lax.rem(nxt, _NBUF))

    slot = jax.lax.rem(i, _NBUF)
    pltpu.make_async_copy(
        fadj_ref.at[pl.ds(i * bm, bm), :],
        buf_ref.at[slot],
        sem_ref.at[slot],
    ).wait()

    a = buf_ref[slot].astype(jnp.bfloat16)
    t = jnp.dot(a, xb_ref[...], preferred_element_type=jnp.float32)
    h = jnp.dot(t, wgc_ref[...], preferred_element_type=jnp.float32)
    h = h + bgc_ref[...]
    h = jnp.where(h > 0, h, jnp.exp(jnp.minimum(h, 0.0)) - 1.0)
    out_ref[...] = (
        jnp.dot(h, wfc_ref[...], preferred_element_type=jnp.float32)
        + bfc_ref[...]
    )


@jax.jit
def kernel(input, fadj, W_gc, b_gc, W_fc, b_fc):
    n, n_in = input.shape
    nfea = W_gc.shape[1]
    n_class = W_fc.shape[1]

    bm = 200
    out = pl.pallas_call(
        _gcn_kernel,
        grid=(n // bm,),
        in_specs=[
            pl.BlockSpec((n, n_in), lambda i: (0, 0)),
            pl.BlockSpec((n_in, nfea), lambda i: (0, 0)),
            pl.BlockSpec((nfea, n_class), lambda i: (0, 0)),
            pl.BlockSpec((1, nfea), lambda i: (0, 0)),
            pl.BlockSpec((1, n_class), lambda i: (0, 0)),
            pl.BlockSpec(memory_space=pltpu.MemorySpace.HBM),
        ],
        out_specs=pl.BlockSpec((bm, n_class), lambda i: (i, 0)),
        out_shape=jax.ShapeDtypeStruct((n, n_class), jnp.float32),
        scratch_shapes=[
            pltpu.VMEM((n, n_in), jnp.bfloat16),
            pltpu.VMEM((_NBUF, bm, n), jnp.float32),
            pltpu.SemaphoreType.DMA((_NBUF,)),
        ],
        compiler_params=pltpu.CompilerParams(
            dimension_semantics=("arbitrary",),
        ),
    )(
        input,
        W_gc,
        W_fc,
        b_gc.reshape(1, nfea),
        b_fc.reshape(1, n_class),
        fadj,
    )

    return out
